# initial kernel scaffold (unmeasured)
import functools

import jax
import jax.numpy as jnp
from jax import lax
from jax.experimental import pallas as pl
from jax.experimental.pallas import tpu as pltpu

N_DEV = 8
M_PER = 512
K = 4096
N_TOT = 8192
N_PER = 1024

E4M3_MAX = 448.0


def _gemm_body(x_ref, w_ref, out_ref, amax_ref):
    j = pl.program_id(0)
    blk = jnp.dot(x_ref[...], w_ref[...], preferred_element_type=jnp.float32)
    out_ref[...] = blk
    bm = jnp.max(jnp.abs(blk))

    @pl.when(j == 0)
    def _():
        amax_ref[0, 0] = bm

    @pl.when(j != 0)
    def _():
        amax_ref[0, 0] = jnp.maximum(amax_ref[0, 0], bm)


def _a2a_body(partial_ref, amax_ref, out_ref,
              amax_buf, send_sems, recv_sems, ax_send_sems, ax_recv_sems):
    my = lax.axis_index("i")

    barrier_sem = pltpu.get_barrier_semaphore()
    for k in range(1, N_DEV):
        dst = lax.rem(my + k, N_DEV)
        pl.semaphore_signal(barrier_sem, inc=1, device_id=(dst,),
                            device_id_type=pl.DeviceIdType.MESH)
    pl.semaphore_wait(barrier_sem, N_DEV - 1)

    a = amax_ref[0, 0]
    amax_buf[pl.ds(my, 1), :] = jnp.full((1, 128), a, jnp.float32)
    ax_sends = []
    for k in range(1, N_DEV):
        dst = lax.rem(my + k, N_DEV)
        r = pltpu.make_async_remote_copy(
            src_ref=amax_buf.at[pl.ds(my, 1)],
            dst_ref=amax_buf.at[pl.ds(my, 1)],
            send_sem=ax_send_sems.at[k],
            recv_sem=ax_recv_sems.at[my],
            device_id=(dst,),
            device_id_type=pl.DeviceIdType.MESH,
        )
        r.start()
        ax_sends.append(r)

    blk_sends = []
    for k in range(1, N_DEV):
        dst = lax.rem(my + k, N_DEV)
        r = pltpu.make_async_remote_copy(
            src_ref=partial_ref.at[:, pl.ds(dst * N_PER, N_PER)],
            dst_ref=out_ref.at[pl.ds(my * M_PER, M_PER), :],
            send_sem=send_sems.at[k],
            recv_sem=recv_sems.at[my],
            device_id=(dst,),
            device_id_type=pl.DeviceIdType.MESH,
        )
        r.start()
        blk_sends.append(r)

    out_ref[pl.ds(my * M_PER, M_PER), :] = (
        partial_ref[:, pl.ds(my * N_PER, N_PER)])

    for k in range(1, N_DEV):
        src = lax.rem(my + N_DEV - k, N_DEV)
        recv = pltpu.make_async_remote_copy(
            src_ref=partial_ref.at[:, pl.ds(0, N_PER)],
            dst_ref=out_ref.at[pl.ds(src * M_PER, M_PER), :],
            send_sem=send_sems.at[0],
            recv_sem=recv_sems.at[src],
            device_id=(src,),
            device_id_type=pl.DeviceIdType.MESH,
        )
        recv.wait_recv()
        ax_recv = pltpu.make_async_remote_copy(
            src_ref=amax_buf.at[pl.ds(src, 1)],
            dst_ref=amax_buf.at[pl.ds(src, 1)],
            send_sem=ax_send_sems.at[0],
            recv_sem=ax_recv_sems.at[src],
            device_id=(src,),
            device_id_type=pl.DeviceIdType.MESH,
        )
        ax_recv.wait_recv()

    for r in ax_sends:
        r.wait_send()
    for r in blk_sends:
        r.wait_send()

    g_amax = jnp.max(amax_buf[:, 0])
    scale = g_amax / E4M3_MAX
    y = out_ref[...]
    q = (y / scale).astype(jnp.float8_e4m3fn).astype(jnp.float32)
    out_ref[...] = q * scale

    @functools.partial(pl.run_scoped, exit_sem=pltpu.SemaphoreType.REGULAR)
    def _(exit_sem):
        for k in range(1, N_DEV):
            dst = lax.rem(my + k, N_DEV)
            pl.semaphore_signal(exit_sem, inc=1, device_id=(dst,),
                                device_id_type=pl.DeviceIdType.MESH)
        pl.semaphore_wait(exit_sem, N_DEV - 1)


def kernel(x, w_mat):
    partial, amax = pl.pallas_call(
        _gemm_body,
        grid=(N_DEV,),
        in_specs=[
            pl.BlockSpec((M_PER, K), lambda j: (0, 0)),
            pl.BlockSpec((K, N_PER), lambda j: (0, j)),
        ],
        out_specs=[
            pl.BlockSpec((M_PER, N_PER), lambda j: (0, j)),
            pl.BlockSpec(memory_space=pltpu.SMEM),
        ],
        out_shape=[
            jax.ShapeDtypeStruct((M_PER, N_TOT), jnp.float32),
            jax.ShapeDtypeStruct((1, 1), jnp.float32),
        ],
    )(x, w_mat)

    return pl.pallas_call(
        _a2a_body,
        in_specs=[
            pl.BlockSpec(memory_space=pltpu.VMEM),
            pl.BlockSpec(memory_space=pltpu.SMEM),
        ],
        out_specs=pl.BlockSpec(memory_space=pltpu.VMEM),
        out_shape=jax.ShapeDtypeStruct((N_DEV * M_PER, N_PER), jnp.float32),
        scratch_shapes=[
            pltpu.VMEM((N_DEV, 128), jnp.float32),
            pltpu.SemaphoreType.DMA((N_DEV,)),
            pltpu.SemaphoreType.DMA((N_DEV,)),
            pltpu.SemaphoreType.DMA((N_DEV,)),
            pltpu.SemaphoreType.DMA((N_DEV,)),
        ],
        compiler_params=pltpu.CompilerParams(collective_id=0),
    )(partial, amax)


# baseline (device time: 224861 ns/iter reference)
import functools

import jax
import jax.numpy as jnp
from jax import lax
from jax.experimental import pallas as pl
from jax.experimental.pallas import tpu as pltpu

N_DEV = 8
M_PER = 512
K = 4096
N_TOT = 8192
N_PER = 1024

E4M3_MAX = 448.0


def _gemm_body(x_ref, w_ref, out_ref, amax_ref):
    j = pl.program_id(0)
    blk = jnp.dot(x_ref[...], w_ref[...], preferred_element_type=jnp.float32)
    out_ref[...] = blk
    bm = jnp.max(jnp.abs(blk))

    @pl.when(j == 0)
    def _():
        amax_ref[0, 0] = bm

    @pl.when(j != 0)
    def _():
        amax_ref[0, 0] = jnp.maximum(amax_ref[0, 0], bm)


def _a2a_body(partial_ref, amax_ref, out_ref,
              amax_buf, send_sems, recv_sems, ax_send_sems, ax_recv_sems):
    my = lax.axis_index("i")

    barrier_sem = pltpu.get_barrier_semaphore()
    for k in range(1, N_DEV):
        dst = lax.rem(my + k, N_DEV)
        pl.semaphore_signal(barrier_sem, inc=1, device_id=(dst,),
                            device_id_type=pl.DeviceIdType.MESH)
    pl.semaphore_wait(barrier_sem, N_DEV - 1)

    a = amax_ref[0, 0]
    amax_buf[pl.ds(my, 1), :] = jnp.full((1, 128), a, jnp.float32)
    ax_sends = []
    for k in range(1, N_DEV):
        dst = lax.rem(my + k, N_DEV)
        r = pltpu.make_async_remote_copy(
            src_ref=amax_buf.at[pl.ds(my, 1)],
            dst_ref=amax_buf.at[pl.ds(my, 1)],
            send_sem=ax_send_sems.at[k],
            recv_sem=ax_recv_sems.at[my],
            device_id=(dst,),
            device_id_type=pl.DeviceIdType.MESH,
        )
        r.start()
        ax_sends.append(r)

    blk_sends = []
    for k in range(1, N_DEV):
        dst = lax.rem(my + k, N_DEV)
        r = pltpu.make_async_remote_copy(
            src_ref=partial_ref.at[:, pl.ds(dst * N_PER, N_PER)],
            dst_ref=out_ref.at[pl.ds(my * M_PER, M_PER), :],
            send_sem=send_sems.at[k],
            recv_sem=recv_sems.at[my],
            device_id=(dst,),
            device_id_type=pl.DeviceIdType.MESH,
        )
        r.start()
        blk_sends.append(r)

    out_ref[pl.ds(my * M_PER, M_PER), :] = (
        partial_ref[:, pl.ds(my * N_PER, N_PER)])

    for k in range(1, N_DEV):
        src = lax.rem(my + N_DEV - k, N_DEV)
        recv = pltpu.make_async_remote_copy(
            src_ref=partial_ref.at[:, pl.ds(0, N_PER)],
            dst_ref=out_ref.at[pl.ds(src * M_PER, M_PER), :],
            send_sem=send_sems.at[0],
            recv_sem=recv_sems.at[src],
            device_id=(src,),
            device_id_type=pl.DeviceIdType.MESH,
        )
        recv.wait_recv()
        ax_recv = pltpu.make_async_remote_copy(
            src_ref=amax_buf.at[pl.ds(src, 1)],
            dst_ref=amax_buf.at[pl.ds(src, 1)],
            send_sem=ax_send_sems.at[0],
            recv_sem=ax_recv_sems.at[src],
            device_id=(src,),
            device_id_type=pl.DeviceIdType.MESH,
        )
        ax_recv.wait_recv()

    for r in ax_sends:
        r.wait_send()
    for r in blk_sends:
        r.wait_send()

    g_amax = jnp.max(amax_buf[:, 0])
    scale = g_amax / E4M3_MAX
    y = out_ref[...]
    q = (y / scale).astype(jnp.float8_e4m3fn).astype(jnp.float32)
    out_ref[...] = q * scale

    @functools.partial(pl.run_scoped, exit_sem=pltpu.SemaphoreType.REGULAR)
    def _(exit_sem):
        for k in range(1, N_DEV):
            dst = lax.rem(my + k, N_DEV)
            pl.semaphore_signal(exit_sem, inc=1, device_id=(dst,),
                                device_id_type=pl.DeviceIdType.MESH)
        pl.semaphore_wait(exit_sem, N_DEV - 1)


def kernel(x, w_mat):
    partial, amax = pl.pallas_call(
        _gemm_body,
        grid=(N_DEV,),
        in_specs=[
            pl.BlockSpec((M_PER, K), lambda j: (0, 0)),
            pl.BlockSpec((K, N_PER), lambda j: (0, j)),
        ],
        out_specs=[
            pl.BlockSpec((M_PER, N_PER), lambda j: (0, j)),
            pl.BlockSpec(memory_space=pltpu.SMEM),
        ],
        out_shape=[
            jax.ShapeDtypeStruct((M_PER, N_TOT), jnp.float32),
            jax.ShapeDtypeStruct((1, 1), jnp.float32),
        ],
        compiler_params=pltpu.CompilerParams(
            vmem_limit_bytes=100 * 1024 * 1024),
    )(x, w_mat)

    return pl.pallas_call(
        _a2a_body,
        in_specs=[
            pl.BlockSpec(memory_space=pltpu.VMEM),
            pl.BlockSpec(memory_space=pltpu.SMEM),
        ],
        out_specs=pl.BlockSpec(memory_space=pltpu.VMEM),
        out_shape=jax.ShapeDtypeStruct((N_DEV * M_PER, N_PER), jnp.float32),
        scratch_shapes=[
            pltpu.VMEM((N_DEV, 128), jnp.float32),
            pltpu.SemaphoreType.DMA((N_DEV,)),
            pltpu.SemaphoreType.DMA((N_DEV,)),
            pltpu.SemaphoreType.DMA((N_DEV,)),
            pltpu.SemaphoreType.DMA((N_DEV,)),
        ],
        compiler_params=pltpu.CompilerParams(
            collective_id=0, vmem_limit_bytes=100 * 1024 * 1024),
    )(partial, amax)


# device time: 125826 ns/iter; 1.7871x vs baseline; 1.7871x over previous
import functools

import jax
import jax.numpy as jnp
from jax import lax
from jax.experimental import pallas as pl
from jax.experimental.pallas import tpu as pltpu

N_DEV = 8
M_PER = 512
K = 4096
N_TOT = 8192
N_PER = 1024

E4M3_MAX = 448.0


def _gemm_body(x_ref, w_ref, out_ref, amax_ref):
    j = pl.program_id(0)
    blk = jnp.dot(x_ref[...], w_ref[...], preferred_element_type=jnp.float32)
    out_ref[...] = blk
    bm = jnp.max(jnp.abs(blk))

    @pl.when(j == 0)
    def _():
        amax_ref[0, 0] = bm

    @pl.when(j != 0)
    def _():
        amax_ref[0, 0] = jnp.maximum(amax_ref[0, 0], bm)


def _a2a_body(partial_ref, amax_ref, out_ref,
              q_send, q_recv, amax_buf,
              send_sems, recv_sems, ax_send_sems, ax_recv_sems):
    my = lax.axis_index("i")

    barrier_sem = pltpu.get_barrier_semaphore()
    for k in range(1, N_DEV):
        dst = lax.rem(my + k, N_DEV)
        pl.semaphore_signal(barrier_sem, inc=1, device_id=(dst,),
                            device_id_type=pl.DeviceIdType.MESH)
    pl.semaphore_wait(barrier_sem, N_DEV - 1)

    a = amax_ref[0, 0]
    amax_buf[pl.ds(my, 1), :] = jnp.full((1, 128), a, jnp.float32)
    ax_sends = []
    for k in range(1, N_DEV):
        dst = lax.rem(my + k, N_DEV)
        r = pltpu.make_async_remote_copy(
            src_ref=amax_buf.at[pl.ds(my, 1)],
            dst_ref=amax_buf.at[pl.ds(my, 1)],
            send_sem=ax_send_sems.at[k],
            recv_sem=ax_recv_sems.at[my],
            device_id=(dst,),
            device_id_type=pl.DeviceIdType.MESH,
        )
        r.start()
        ax_sends.append(r)

    for k in range(1, N_DEV):
        src = lax.rem(my + N_DEV - k, N_DEV)
        ax_recv = pltpu.make_async_remote_copy(
            src_ref=amax_buf.at[pl.ds(src, 1)],
            dst_ref=amax_buf.at[pl.ds(src, 1)],
            send_sem=ax_send_sems.at[0],
            recv_sem=ax_recv_sems.at[src],
            device_id=(src,),
            device_id_type=pl.DeviceIdType.MESH,
        )
        ax_recv.wait_recv()
    g_amax = jnp.max(amax_buf[:, 0])
    scale = g_amax / E4M3_MAX
    inv_scale = E4M3_MAX / g_amax

    q_send[...] = (partial_ref[...] * inv_scale).astype(jnp.float8_e4m3fn)

    blk_sends = []
    for k in range(1, N_DEV):
        dst = lax.rem(my + k, N_DEV)
        r = pltpu.make_async_remote_copy(
            src_ref=q_send.at[:, pl.ds(dst * N_PER, N_PER)],
            dst_ref=q_recv.at[pl.ds(my * M_PER, M_PER), :],
            send_sem=send_sems.at[k],
            recv_sem=recv_sems.at[my],
            device_id=(dst,),
            device_id_type=pl.DeviceIdType.MESH,
        )
        r.start()
        blk_sends.append(r)

    q_recv[pl.ds(my * M_PER, M_PER), :] = (
        q_send[:, pl.ds(my * N_PER, N_PER)])

    for k in range(1, N_DEV):
        src = lax.rem(my + N_DEV - k, N_DEV)
        recv = pltpu.make_async_remote_copy(
            src_ref=q_send.at[:, pl.ds(0, N_PER)],
            dst_ref=q_recv.at[pl.ds(src * M_PER, M_PER), :],
            send_sem=send_sems.at[0],
            recv_sem=recv_sems.at[src],
            device_id=(src,),
            device_id_type=pl.DeviceIdType.MESH,
        )
        recv.wait_recv()

    for r in ax_sends:
        r.wait_send()
    for r in blk_sends:
        r.wait_send()

    out_ref[...] = q_recv[...].astype(jnp.float32) * scale

    @functools.partial(pl.run_scoped, exit_sem=pltpu.SemaphoreType.REGULAR)
    def _(exit_sem):
        for k in range(1, N_DEV):
            dst = lax.rem(my + k, N_DEV)
            pl.semaphore_signal(exit_sem, inc=1, device_id=(dst,),
                                device_id_type=pl.DeviceIdType.MESH)
        pl.semaphore_wait(exit_sem, N_DEV - 1)


def kernel(x, w_mat):
    partial, amax = pl.pallas_call(
        _gemm_body,
        grid=(N_DEV,),
        in_specs=[
            pl.BlockSpec((M_PER, K), lambda j: (0, 0)),
            pl.BlockSpec((K, N_PER), lambda j: (0, j)),
        ],
        out_specs=[
            pl.BlockSpec((M_PER, N_PER), lambda j: (0, j)),
            pl.BlockSpec(memory_space=pltpu.SMEM),
        ],
        out_shape=[
            jax.ShapeDtypeStruct((M_PER, N_TOT), jnp.float32),
            jax.ShapeDtypeStruct((1, 1), jnp.float32),
        ],
        compiler_params=pltpu.CompilerParams(
            vmem_limit_bytes=100 * 1024 * 1024),
    )(x, w_mat)

    return pl.pallas_call(
        _a2a_body,
        in_specs=[
            pl.BlockSpec(memory_space=pltpu.VMEM),
            pl.BlockSpec(memory_space=pltpu.SMEM),
        ],
        out_specs=pl.BlockSpec(memory_space=pltpu.VMEM),
        out_shape=jax.ShapeDtypeStruct((N_DEV * M_PER, N_PER), jnp.float32),
        scratch_shapes=[
            pltpu.VMEM((M_PER, N_TOT), jnp.float8_e4m3fn),
            pltpu.VMEM((N_DEV * M_PER, N_PER), jnp.float8_e4m3fn),
            pltpu.VMEM((N_DEV, 128), jnp.float32),
            pltpu.SemaphoreType.DMA((N_DEV,)),
            pltpu.SemaphoreType.DMA((N_DEV,)),
            pltpu.SemaphoreType.DMA((N_DEV,)),
            pltpu.SemaphoreType.DMA((N_DEV,)),
        ],
        compiler_params=pltpu.CompilerParams(
            collective_id=0, vmem_limit_bytes=100 * 1024 * 1024),
    )(partial, amax)


# device time: 119641 ns/iter; 1.8795x vs baseline; 1.0517x over previous
import functools

import jax
import jax.numpy as jnp
from jax import lax
from jax.experimental import pallas as pl
from jax.experimental.pallas import tpu as pltpu

N_DEV = 8
M_PER = 512
K = 4096
N_TOT = 8192
N_PER = 1024

E4M3_MAX = 448.0


def _gemm_body(x_ref, w_ref, out_ref, amax_ref):
    j = pl.program_id(0)
    blk = jnp.dot(x_ref[...], w_ref[...], preferred_element_type=jnp.float32)
    out_ref[...] = blk
    bm = jnp.max(jnp.abs(blk))

    @pl.when(j == 0)
    def _():
        amax_ref[0, 0] = bm

    @pl.when(j != 0)
    def _():
        amax_ref[0, 0] = jnp.maximum(amax_ref[0, 0], bm)


def _a2a_body(partial_hbm, amax_ref, out_ref,
              stage, q_send, q_recv, amax_buf,
              copy_sems, send_sems, recv_sems, ax_send_sems, ax_recv_sems):
    my = lax.axis_index("i")

    barrier_sem = pltpu.get_barrier_semaphore()
    for k in range(1, N_DEV):
        dst = lax.rem(my + k, N_DEV)
        pl.semaphore_signal(barrier_sem, inc=1, device_id=(dst,),
                            device_id_type=pl.DeviceIdType.MESH)
    pl.semaphore_wait(barrier_sem, N_DEV - 1)

    a = amax_ref[0, 0]
    amax_buf[pl.ds(my, 1), :] = jnp.full((1, 128), a, jnp.float32)
    ax_sends = []
    for k in range(1, N_DEV):
        dst = lax.rem(my + k, N_DEV)
        r = pltpu.make_async_remote_copy(
            src_ref=amax_buf.at[pl.ds(my, 1)],
            dst_ref=amax_buf.at[pl.ds(my, 1)],
            send_sem=ax_send_sems.at[k],
            recv_sem=ax_recv_sems.at[my],
            device_id=(dst,),
            device_id_type=pl.DeviceIdType.MESH,
        )
        r.start()
        ax_sends.append(r)

    for k in range(1, N_DEV):
        src = lax.rem(my + N_DEV - k, N_DEV)
        ax_recv = pltpu.make_async_remote_copy(
            src_ref=amax_buf.at[pl.ds(src, 1)],
            dst_ref=amax_buf.at[pl.ds(src, 1)],
            send_sem=ax_send_sems.at[0],
            recv_sem=ax_recv_sems.at[src],
            device_id=(src,),
            device_id_type=pl.DeviceIdType.MESH,
        )
        ax_recv.wait_recv()
    g_amax = jnp.max(amax_buf[:, 0])
    scale = g_amax / E4M3_MAX
    inv_scale = E4M3_MAX / g_amax

    def _readback(k, slot):
        dst = lax.rem(my + k, N_DEV)
        cp = pltpu.make_async_copy(
            src_ref=partial_hbm.at[:, pl.ds(dst * N_PER, N_PER)],
            dst_ref=stage.at[slot],
            sem=copy_sems.at[slot],
        )
        cp.start()
        return cp

    cps = [None, None]
    cps[1] = _readback(1, 1)
    blk_sends = []
    for k in range(1, N_DEV):
        slot = k % 2
        nxt = (k + 1) % 2
        if k < N_DEV - 1:
            cps[nxt] = _readback(k + 1, nxt)
        else:
            cps[nxt] = _readback(0, nxt)
        cps[slot].wait()
        dst = lax.rem(my + k, N_DEV)
        q_send[:, pl.ds(dst * N_PER, N_PER)] = (
            stage[slot] * inv_scale).astype(jnp.float8_e4m3fn)
        r = pltpu.make_async_remote_copy(
            src_ref=q_send.at[:, pl.ds(dst * N_PER, N_PER)],
            dst_ref=q_recv.at[pl.ds(my * M_PER, M_PER), :],
            send_sem=send_sems.at[k],
            recv_sem=recv_sems.at[my],
            device_id=(dst,),
            device_id_type=pl.DeviceIdType.MESH,
        )
        r.start()
        blk_sends.append(r)

    cps[0].wait()
    own_q = (stage[0] * inv_scale).astype(jnp.float8_e4m3fn)
    out_ref[pl.ds(my * M_PER, M_PER), :] = own_q.astype(jnp.float32) * scale

    for k in range(1, N_DEV):
        src = lax.rem(my + N_DEV - k, N_DEV)
        recv = pltpu.make_async_remote_copy(
            src_ref=q_send.at[:, pl.ds(0, N_PER)],
            dst_ref=q_recv.at[pl.ds(src * M_PER, M_PER), :],
            send_sem=send_sems.at[0],
            recv_sem=recv_sems.at[src],
            device_id=(src,),
            device_id_type=pl.DeviceIdType.MESH,
        )
        recv.wait_recv()
        out_ref[pl.ds(src * M_PER, M_PER), :] = (
            q_recv[pl.ds(src * M_PER, M_PER), :].astype(jnp.float32) * scale)

    for r in ax_sends:
        r.wait_send()
    for r in blk_sends:
        r.wait_send()

    @functools.partial(pl.run_scoped, exit_sem=pltpu.SemaphoreType.REGULAR)
    def _(exit_sem):
        for k in range(1, N_DEV):
            dst = lax.rem(my + k, N_DEV)
            pl.semaphore_signal(exit_sem, inc=1, device_id=(dst,),
                                device_id_type=pl.DeviceIdType.MESH)
        pl.semaphore_wait(exit_sem, N_DEV - 1)


def kernel(x, w_mat):
    partial, amax = pl.pallas_call(
        _gemm_body,
        grid=(N_DEV,),
        in_specs=[
            pl.BlockSpec((M_PER, K), lambda j: (0, 0)),
            pl.BlockSpec((K, N_PER), lambda j: (0, j)),
        ],
        out_specs=[
            pl.BlockSpec((M_PER, N_PER), lambda j: (0, j)),
            pl.BlockSpec(memory_space=pltpu.SMEM),
        ],
        out_shape=[
            jax.ShapeDtypeStruct((M_PER, N_TOT), jnp.float32),
            jax.ShapeDtypeStruct((1, 1), jnp.float32),
        ],
        compiler_params=pltpu.CompilerParams(
            vmem_limit_bytes=100 * 1024 * 1024),
    )(x, w_mat)

    return pl.pallas_call(
        _a2a_body,
        in_specs=[
            pl.BlockSpec(memory_space=pl.ANY),
            pl.BlockSpec(memory_space=pltpu.SMEM),
        ],
        out_specs=pl.BlockSpec(memory_space=pltpu.VMEM),
        out_shape=jax.ShapeDtypeStruct((N_DEV * M_PER, N_PER), jnp.float32),
        scratch_shapes=[
            pltpu.VMEM((2, M_PER, N_PER), jnp.float32),
            pltpu.VMEM((M_PER, N_TOT), jnp.float8_e4m3fn),
            pltpu.VMEM((N_DEV * M_PER, N_PER), jnp.float8_e4m3fn),
            pltpu.VMEM((N_DEV, 128), jnp.float32),
            pltpu.SemaphoreType.DMA((2,)),
            pltpu.SemaphoreType.DMA((N_DEV,)),
            pltpu.SemaphoreType.DMA((N_DEV,)),
            pltpu.SemaphoreType.DMA((N_DEV,)),
            pltpu.SemaphoreType.DMA((N_DEV,)),
        ],
        compiler_params=pltpu.CompilerParams(
            collective_id=0, vmem_limit_bytes=100 * 1024 * 1024),
    )(partial, amax)


# device time: 113605 ns/iter; 1.9793x vs baseline; 1.0531x over previous
import functools

import jax
import jax.numpy as jnp
from jax import lax
from jax.experimental import pallas as pl
from jax.experimental.pallas import tpu as pltpu

N_DEV = 8
M_PER = 512
K = 4096
N_TOT = 8192
N_PER = 1024

E4M3_MAX = 448.0


def _gemm_body(x_ref, w_ref, out_ref, amax_ref):
    j = pl.program_id(0)
    blk = jnp.dot(x_ref[...], w_ref[...], preferred_element_type=jnp.float32)
    out_ref[...] = blk
    bm = jnp.max(jnp.abs(blk))

    @pl.when(j == 0)
    def _():
        amax_ref[0, 0] = bm

    @pl.when(j != 0)
    def _():
        amax_ref[0, 0] = jnp.maximum(amax_ref[0, 0], bm)


def _a2a_body(partial_hbm, amax_ref, q_out, scale_ref,
              stage, q_send, amax_buf,
              copy_sems, send_sems, recv_sems, ax_send_sems, ax_recv_sems):
    my = lax.axis_index("i")

    barrier_sem = pltpu.get_barrier_semaphore()
    for k in range(1, N_DEV):
        dst = lax.rem(my + k, N_DEV)
        pl.semaphore_signal(barrier_sem, inc=1, device_id=(dst,),
                            device_id_type=pl.DeviceIdType.MESH)
    pl.semaphore_wait(barrier_sem, N_DEV - 1)

    a = amax_ref[0, 0]
    amax_buf[pl.ds(my, 1), :] = jnp.full((1, 128), a, jnp.float32)
    ax_sends = []
    for k in range(1, N_DEV):
        dst = lax.rem(my + k, N_DEV)
        r = pltpu.make_async_remote_copy(
            src_ref=amax_buf.at[pl.ds(my, 1)],
            dst_ref=amax_buf.at[pl.ds(my, 1)],
            send_sem=ax_send_sems.at[k],
            recv_sem=ax_recv_sems.at[my],
            device_id=(dst,),
            device_id_type=pl.DeviceIdType.MESH,
        )
        r.start()
        ax_sends.append(r)

    for k in range(1, N_DEV):
        src = lax.rem(my + N_DEV - k, N_DEV)
        ax_recv = pltpu.make_async_remote_copy(
            src_ref=amax_buf.at[pl.ds(src, 1)],
            dst_ref=amax_buf.at[pl.ds(src, 1)],
            send_sem=ax_send_sems.at[0],
            recv_sem=ax_recv_sems.at[src],
            device_id=(src,),
            device_id_type=pl.DeviceIdType.MESH,
        )
        ax_recv.wait_recv()
    g_amax = jnp.max(amax_buf[:, 0])
    scale_ref[0, 0] = g_amax / E4M3_MAX
    inv_scale = E4M3_MAX / g_amax

    def _readback(k, slot):
        dst = lax.rem(my + k, N_DEV)
        cp = pltpu.make_async_copy(
            src_ref=partial_hbm.at[:, pl.ds(dst * N_PER, N_PER)],
            dst_ref=stage.at[slot],
            sem=copy_sems.at[slot],
        )
        cp.start()
        return cp

    cps = [None, None]
    cps[1] = _readback(1, 1)
    blk_sends = []
    for k in range(1, N_DEV):
        slot = k % 2
        nxt = (k + 1) % 2
        if k < N_DEV - 1:
            cps[nxt] = _readback(k + 1, nxt)
        else:
            cps[nxt] = _readback(0, nxt)
        cps[slot].wait()
        dst = lax.rem(my + k, N_DEV)
        q_send[:, pl.ds(dst * N_PER, N_PER)] = (
            stage[slot] * inv_scale).astype(jnp.float8_e4m3fn)
        r = pltpu.make_async_remote_copy(
            src_ref=q_send.at[:, pl.ds(dst * N_PER, N_PER)],
            dst_ref=q_out.at[pl.ds(my * M_PER, M_PER), :],
            send_sem=send_sems.at[k],
            recv_sem=recv_sems.at[my],
            device_id=(dst,),
            device_id_type=pl.DeviceIdType.MESH,
        )
        r.start()
        blk_sends.append(r)

    cps[0].wait()
    q_out[pl.ds(my * M_PER, M_PER), :] = (
        stage[0] * inv_scale).astype(jnp.float8_e4m3fn)

    for k in range(1, N_DEV):
        src = lax.rem(my + N_DEV - k, N_DEV)
        recv = pltpu.make_async_remote_copy(
            src_ref=q_send.at[:, pl.ds(0, N_PER)],
            dst_ref=q_out.at[pl.ds(src * M_PER, M_PER), :],
            send_sem=send_sems.at[0],
            recv_sem=recv_sems.at[src],
            device_id=(src,),
            device_id_type=pl.DeviceIdType.MESH,
        )
        recv.wait_recv()

    for r in ax_sends:
        r.wait_send()
    for r in blk_sends:
        r.wait_send()

    @functools.partial(pl.run_scoped, exit_sem=pltpu.SemaphoreType.REGULAR)
    def _(exit_sem):
        for k in range(1, N_DEV):
            dst = lax.rem(my + k, N_DEV)
            pl.semaphore_signal(exit_sem, inc=1, device_id=(dst,),
                                device_id_type=pl.DeviceIdType.MESH)
        pl.semaphore_wait(exit_sem, N_DEV - 1)


def kernel(x, w_mat):
    partial, amax = pl.pallas_call(
        _gemm_body,
        grid=(N_DEV,),
        in_specs=[
            pl.BlockSpec((M_PER, K), lambda j: (0, 0)),
            pl.BlockSpec((K, N_PER), lambda j: (0, j)),
        ],
        out_specs=[
            pl.BlockSpec((M_PER, N_PER), lambda j: (0, j)),
            pl.BlockSpec(memory_space=pltpu.SMEM),
        ],
        out_shape=[
            jax.ShapeDtypeStruct((M_PER, N_TOT), jnp.float32),
            jax.ShapeDtypeStruct((1, 1), jnp.float32),
        ],
        compiler_params=pltpu.CompilerParams(
            vmem_limit_bytes=100 * 1024 * 1024),
    )(x, w_mat)

    q, scale = pl.pallas_call(
        _a2a_body,
        in_specs=[
            pl.BlockSpec(memory_space=pl.ANY),
            pl.BlockSpec(memory_space=pltpu.SMEM),
        ],
        out_specs=[
            pl.BlockSpec(memory_space=pltpu.VMEM),
            pl.BlockSpec(memory_space=pltpu.SMEM),
        ],
        out_shape=[
            jax.ShapeDtypeStruct((N_DEV * M_PER, N_PER), jnp.float8_e4m3fn),
            jax.ShapeDtypeStruct((1, 1), jnp.float32),
        ],
        scratch_shapes=[
            pltpu.VMEM((2, M_PER, N_PER), jnp.float32),
            pltpu.VMEM((M_PER, N_TOT), jnp.float8_e4m3fn),
            pltpu.VMEM((N_DEV, 128), jnp.float32),
            pltpu.SemaphoreType.DMA((2,)),
            pltpu.SemaphoreType.DMA((N_DEV,)),
            pltpu.SemaphoreType.DMA((N_DEV,)),
            pltpu.SemaphoreType.DMA((N_DEV,)),
            pltpu.SemaphoreType.DMA((N_DEV,)),
        ],
        compiler_params=pltpu.CompilerParams(
            collective_id=0, vmem_limit_bytes=100 * 1024 * 1024),
    )(partial, amax)
    return q.astype(jnp.float32) * scale[0, 0]


# device time: 108145 ns/iter; 2.0793x vs baseline; 1.0505x over previous
import functools

import jax
import jax.numpy as jnp
from jax import lax
from jax.experimental import pallas as pl
from jax.experimental.pallas import tpu as pltpu

N_DEV = 8
M_PER = 512
K = 4096
N_TOT = 8192
N_PER = 1024
KC = 512
N_CHUNK = N_TOT // KC

E4M3_MAX = 448.0


def _body(x_ref, w_hbm, q_out, scale_ref,
          w_buf, partial, stage_unused, q_send, amax_buf,
          w_sems, copy_sems, send_sems, recv_sems, ax_send_sems,
          ax_recv_sems):
    my = lax.axis_index("i")

    barrier_sem = pltpu.get_barrier_semaphore()
    for k in range(1, N_DEV):
        dst = lax.rem(my + k, N_DEV)
        pl.semaphore_signal(barrier_sem, inc=1, device_id=(dst,),
                            device_id_type=pl.DeviceIdType.MESH)
    pl.semaphore_wait(barrier_sem, N_DEV - 1)

    def _fetch(c, slot):
        cp = pltpu.make_async_copy(
            src_ref=w_hbm.at[:, pl.ds(c * KC, KC)],
            dst_ref=w_buf.at[slot],
            sem=w_sems.at[slot],
        )
        cp.start()
        return cp

    cps = [None, None]
    cps[0] = _fetch(0, 0)
    am = jnp.float32(0.0)
    for c in range(N_CHUNK):
        slot = c % 2
        if c + 1 < N_CHUNK:
            cps[(c + 1) % 2] = _fetch(c + 1, (c + 1) % 2)
        cps[slot].wait()
        blk = jnp.dot(x_ref[...], w_buf[slot],
                      preferred_element_type=jnp.float32)
        partial[:, pl.ds(c * KC, KC)] = blk
        am = jnp.maximum(am, jnp.max(jnp.abs(blk)))

    amax_buf[pl.ds(my, 1), :] = jnp.full((1, 128), am, jnp.float32)
    ax_sends = []
    for k in range(1, N_DEV):
        dst = lax.rem(my + k, N_DEV)
        r = pltpu.make_async_remote_copy(
            src_ref=amax_buf.at[pl.ds(my, 1)],
            dst_ref=amax_buf.at[pl.ds(my, 1)],
            send_sem=ax_send_sems.at[k],
            recv_sem=ax_recv_sems.at[my],
            device_id=(dst,),
            device_id_type=pl.DeviceIdType.MESH,
        )
        r.start()
        ax_sends.append(r)
    for k in range(1, N_DEV):
        src = lax.rem(my + N_DEV - k, N_DEV)
        ax_recv = pltpu.make_async_remote_copy(
            src_ref=amax_buf.at[pl.ds(src, 1)],
            dst_ref=amax_buf.at[pl.ds(src, 1)],
            send_sem=ax_send_sems.at[0],
            recv_sem=ax_recv_sems.at[src],
            device_id=(src,),
            device_id_type=pl.DeviceIdType.MESH,
        )
        ax_recv.wait_recv()
    g_amax = jnp.max(amax_buf[:, 0])
    scale_ref[0, 0] = g_amax / E4M3_MAX
    inv_scale = E4M3_MAX / g_amax

    blk_sends = []
    for k in range(1, N_DEV):
        dst = lax.rem(my + k, N_DEV)
        q_send[:, pl.ds(dst * N_PER, N_PER)] = (
            partial[:, pl.ds(dst * N_PER, N_PER)] * inv_scale
        ).astype(jnp.float8_e4m3fn)
        r = pltpu.make_async_remote_copy(
            src_ref=q_send.at[:, pl.ds(dst * N_PER, N_PER)],
            dst_ref=q_out.at[pl.ds(my * M_PER, M_PER), :],
            send_sem=send_sems.at[k],
            recv_sem=recv_sems.at[my],
            device_id=(dst,),
            device_id_type=pl.DeviceIdType.MESH,
        )
        r.start()
        blk_sends.append(r)

    q_out[pl.ds(my * M_PER, M_PER), :] = (
        partial[:, pl.ds(my * N_PER, N_PER)] * inv_scale
    ).astype(jnp.float8_e4m3fn)

    for k in range(1, N_DEV):
        src = lax.rem(my + N_DEV - k, N_DEV)
        recv = pltpu.make_async_remote_copy(
            src_ref=q_send.at[:, pl.ds(0, N_PER)],
            dst_ref=q_out.at[pl.ds(src * M_PER, M_PER), :],
            send_sem=send_sems.at[0],
            recv_sem=recv_sems.at[src],
            device_id=(src,),
            device_id_type=pl.DeviceIdType.MESH,
        )
        recv.wait_recv()

    for r in ax_sends:
        r.wait_send()
    for r in blk_sends:
        r.wait_send()

    @functools.partial(pl.run_scoped, exit_sem=pltpu.SemaphoreType.REGULAR)
    def _(exit_sem):
        for k in range(1, N_DEV):
            dst = lax.rem(my + k, N_DEV)
            pl.semaphore_signal(exit_sem, inc=1, device_id=(dst,),
                                device_id_type=pl.DeviceIdType.MESH)
        pl.semaphore_wait(exit_sem, N_DEV - 1)


def kernel(x, w_mat):
    q, scale = pl.pallas_call(
        _body,
        in_specs=[
            pl.BlockSpec(memory_space=pltpu.VMEM),
            pl.BlockSpec(memory_space=pl.ANY),
        ],
        out_specs=[
            pl.BlockSpec(memory_space=pltpu.VMEM),
            pl.BlockSpec(memory_space=pltpu.SMEM),
        ],
        out_shape=[
            jax.ShapeDtypeStruct((N_DEV * M_PER, N_PER), jnp.float8_e4m3fn),
            jax.ShapeDtypeStruct((1, 1), jnp.float32),
        ],
        scratch_shapes=[
            pltpu.VMEM((2, K, KC), jnp.float32),
            pltpu.VMEM((M_PER, N_TOT), jnp.float32),
            pltpu.VMEM((8, 128), jnp.float32),
            pltpu.VMEM((M_PER, N_TOT), jnp.float8_e4m3fn),
            pltpu.VMEM((N_DEV, 128), jnp.float32),
            pltpu.SemaphoreType.DMA((2,)),
            pltpu.SemaphoreType.DMA((2,)),
            pltpu.SemaphoreType.DMA((N_DEV,)),
            pltpu.SemaphoreType.DMA((N_DEV,)),
            pltpu.SemaphoreType.DMA((N_DEV,)),
            pltpu.SemaphoreType.DMA((N_DEV,)),
        ],
        compiler_params=pltpu.CompilerParams(
            collective_id=0, vmem_limit_bytes=100 * 1024 * 1024),
    )(x, w_mat)
    return q.astype(jnp.float32) * scale[0, 0]
